# baseline (speedup 1.0000x reference)
import functools

import jax
import jax.numpy as jnp
from jax import lax
from jax.experimental import pallas as pl
from jax.experimental.pallas import tpu as pltpu
from jax.experimental.pallas import tpu_sc as plsc

B = 4096
D = 128
L = 50
LP = 56
NC = 2
NS = 16
NW = NC * NS
GB = 32
NBUF = 6
LCHUNK = 112
NLANE = 16
BH = B // 2


def _make_sc_embed(bh):
    bpw = bh // NW
    ng = bpw // GB

    @functools.partial(
        pl.kernel,
        mesh=plsc.VectorSubcoreMesh(core_axis_name="c", subcore_axis_name="s"),
        compiler_params=pltpu.CompilerParams(needs_layout_passes=False),
        out_type=[
            jax.ShapeDtypeStruct((bh, D), jnp.float32),
            jax.ShapeDtypeStruct((bh, D), jnp.float32),
            jax.ShapeDtypeStruct((bh,), jnp.float32),
        ],
        scratch_types=[
            pltpu.VMEM((GB * LP,), jnp.int32),
            pltpu.VMEM((GB * LP,), jnp.float32),
            pltpu.VMEM((GB * LP,), jnp.float32),
            pltpu.VMEM((NBUF, L, D), jnp.float32),
            pltpu.VMEM((bpw,), jnp.int32),
            pltpu.VMEM((bpw, D), jnp.float32),
            pltpu.VMEM((bpw,), jnp.float32),
            pltpu.VMEM((GB, D), jnp.float32),
            pltpu.VMEM((bpw,), jnp.float32),
            [pltpu.SemaphoreType.DMA] * NBUF,
            pltpu.SemaphoreType.DMA,
            pltpu.SemaphoreType.DMA,
            pltpu.SemaphoreType.DMA,
        ],
    )
    def _sc_embed(ids_hbm, rat_hbm, item_hbm, table_hbm, lin_hbm,
                  user_out, item_out, lin_out,
                  idx_v, rat_v, linv_v, rows_v, item_idx_v, item_rows_v,
                  lin_item_v, user_stage, lin_stage,
                  row_sems, sem_lin, sem_item, sem_lini):
        wid = lax.axis_index("s") * NC + lax.axis_index("c")
        base = wid * bpw

        pltpu.sync_copy(item_hbm.at[pl.ds(base, bpw)], item_idx_v)
        item_cp = pltpu.async_copy(table_hbm.at[item_idx_v], item_rows_v,
                                   sem_item)
        lini_cp = pltpu.async_copy(lin_hbm.at[item_idx_v], lin_item_v,
                                   sem_lini)

        def group_body(g, _):
            gflat = (base + g * GB) * LP
            pltpu.sync_copy(ids_hbm.at[pl.ds(gflat, GB * LP)], idx_v)
            pltpu.sync_copy(rat_hbm.at[pl.ds(gflat, GB * LP)], rat_v)

            lin_cps = []
            for s in range(GB * LP // LCHUNK):
                sl = pl.ds(s * LCHUNK, LCHUNK)
                lin_cps.append(
                    pltpu.async_copy(lin_hbm.at[idx_v.at[sl]], linv_v.at[sl],
                                     sem_lin))

            def fire(r):
                return pltpu.async_copy(
                    table_hbm.at[idx_v.at[pl.ds(r * LP, L)]],
                    rows_v.at[r % NBUF], row_sems[r % NBUF])

            cps = {r: fire(r) for r in range(NBUF - 1)}
            for r in range(GB):
                if r + NBUF - 1 < GB:
                    cps[r + NBUF - 1] = fire(r + NBUF - 1)
                cps[r].wait()
                rows = rows_v.at[r % NBUF]
                lbase = r * LP

                def l_body(l, accs, lbase=lbase, rows=rows):
                    rb = plsc.load_gather(
                        rat_v, [jnp.zeros((NLANE,), jnp.int32) + (lbase + l)])
                    return tuple(
                        accs[j] + rb * rows[l, pl.ds(j * NLANE, NLANE)]
                        for j in range(D // NLANE))

                accs = lax.fori_loop(
                    0, L, l_body,
                    tuple(jnp.zeros((NLANE,), jnp.float32)
                          for _ in range(D // NLANE)))
                for j in range(D // NLANE):
                    user_stage[r, pl.ds(j * NLANE, NLANE)] = accs[j]

            for cp in lin_cps:
                cp.wait()

            for c in range(GB // NLANE):
                lanes = (lax.iota(jnp.int32, NLANE) + c * NLANE) * LP

                def lin_body(l, acc, lanes=lanes):
                    lv = plsc.load_gather(linv_v, [lanes + l])
                    rv = plsc.load_gather(rat_v, [lanes + l])
                    return acc + lv * rv

                lin_acc = lax.fori_loop(0, L, lin_body,
                                        jnp.zeros((NLANE,), jnp.float32))
                lin_stage[pl.ds(g * GB + c * NLANE, NLANE)] = lin_acc

            pltpu.sync_copy(user_stage, user_out.at[pl.ds(base + g * GB, GB)])
            return 0

        lax.fori_loop(0, ng, group_body, 0)

        item_cp.wait()
        pltpu.sync_copy(item_rows_v, item_out.at[pl.ds(base, bpw)])
        lini_cp.wait()
        for c in range(bpw // NLANE):
            sl = pl.ds(c * NLANE, NLANE)
            lin_stage[sl] = lin_stage[sl] + lin_item_v[sl]
        pltpu.sync_copy(lin_stage, lin_out.at[pl.ds(base, bpw)])

    return _sc_embed


_sc_embed_half = _make_sc_embed(BH)

BT = 512


def _tc_body(u_ref, i_ref, lin_ref, w0, b0, w1, b1, w2, b2, w3t, bias, o_ref):
    u = u_ref[...]
    it = i_ref[...]
    x = jnp.concatenate([u, it], axis=1)
    h = jnp.maximum(jnp.dot(x, w0[...], preferred_element_type=jnp.float32)
                    + b0[...], 0.0)
    h = jnp.maximum(jnp.dot(h, w1[...], preferred_element_type=jnp.float32)
                    + b1[...], 0.0)
    h = jnp.maximum(jnp.dot(h, w2[...], preferred_element_type=jnp.float32)
                    + b2[...], 0.0)
    mlp = jnp.sum(h * w3t[...], axis=1)
    fm = jnp.sum(u * it, axis=1)
    z = lin_ref[...] + fm + mlp + bias[0, 0]
    o_ref[...] = 1.0 / (1.0 + jnp.exp(-z))


def _tc_mlp(user, item, lin, w0, b0, w1, b1, w2, b2, w3t, bias):
    bh = user.shape[0]
    grid = (bh // BT,)
    full = lambda r, c: pl.BlockSpec((r, c), lambda i: (0, 0))
    return pl.pallas_call(
        _tc_body,
        grid=grid,
        in_specs=[
            pl.BlockSpec((BT, D), lambda i: (i, 0)),
            pl.BlockSpec((BT, D), lambda i: (i, 0)),
            pl.BlockSpec((BT,), lambda i: (i,)),
            full(2 * D, 1024),
            full(1, 1024),
            full(1024, 512),
            full(1, 512),
            full(512, 256),
            full(1, 256),
            full(1, 256),
            pl.BlockSpec(memory_space=pltpu.SMEM),
        ],
        out_specs=pl.BlockSpec((BT,), lambda i: (i,)),
        out_shape=jax.ShapeDtypeStruct((bh,), jnp.float32),
        compiler_params=pltpu.CompilerParams(
            dimension_semantics=("arbitrary",)),
    )(user, item, lin, w0, b0, w1, b1, w2, b2, w3t, bias)


def kernel(feature_ids, feature_ratings, item_ids, emb_table, lin_table,
           lin_bias, W0, b0, W1, b1, W2, b2, W3, b3):
    ids = jnp.pad(feature_ids.astype(jnp.int32),
                  ((0, 0), (0, LP - L))).reshape(-1)
    rat = jnp.pad(feature_ratings, ((0, 0), (0, LP - L))).reshape(-1)
    itm = item_ids.astype(jnp.int32)
    lin_flat = lin_table[:, 0]
    halves = []
    for h in range(2):
        halves.append(_sc_embed_half(
            ids[h * BH * LP:(h + 1) * BH * LP],
            rat[h * BH * LP:(h + 1) * BH * LP],
            itm[h * BH:(h + 1) * BH], emb_table, lin_flat))
    bias = (lin_bias + b3).reshape(1, 1)
    outs = [
        _tc_mlp(u, i, lin, W0, b0.reshape(1, -1), W1, b1.reshape(1, -1),
                W2, b2.reshape(1, -1), W3.reshape(1, -1), bias)
        for (u, i, lin) in halves
    ]
    return jnp.concatenate(outs)


# speedup vs baseline: 1.0783x; 1.0783x over previous
import functools

import jax
import jax.numpy as jnp
from jax import lax
from jax.experimental import pallas as pl
from jax.experimental.pallas import tpu as pltpu
from jax.experimental.pallas import tpu_sc as plsc

B = 4096
D = 128
L = 50
LP = 56
NC = 2
NS = 16
NW = NC * NS
GB = 32
NBUF = 6
LCHUNK = 112
NLANE = 16
BH = B


def _make_sc_embed(bh):
    bpw = bh // NW
    ng = bpw // GB

    @functools.partial(
        pl.kernel,
        mesh=plsc.VectorSubcoreMesh(core_axis_name="c", subcore_axis_name="s"),
        compiler_params=pltpu.CompilerParams(needs_layout_passes=False),
        out_type=[
            jax.ShapeDtypeStruct((bh, D), jnp.float32),
            jax.ShapeDtypeStruct((bh, D), jnp.float32),
            jax.ShapeDtypeStruct((bh,), jnp.float32),
        ],
        scratch_types=[
            pltpu.VMEM((GB * LP,), jnp.int32),
            pltpu.VMEM((GB * LP,), jnp.float32),
            pltpu.VMEM((GB * LP,), jnp.float32),
            pltpu.VMEM((NBUF, L, D), jnp.float32),
            pltpu.VMEM((bpw,), jnp.int32),
            pltpu.VMEM((bpw, D), jnp.float32),
            pltpu.VMEM((bpw,), jnp.float32),
            pltpu.VMEM((GB, D), jnp.float32),
            pltpu.VMEM((bpw,), jnp.float32),
            [pltpu.SemaphoreType.DMA] * NBUF,
            pltpu.SemaphoreType.DMA,
            pltpu.SemaphoreType.DMA,
            pltpu.SemaphoreType.DMA,
        ],
    )
    def _sc_embed(ids_hbm, rat_hbm, item_hbm, table_hbm, lin_hbm,
                  user_out, item_out, lin_out,
                  idx_v, rat_v, linv_v, rows_v, item_idx_v, item_rows_v,
                  lin_item_v, user_stage, lin_stage,
                  row_sems, sem_lin, sem_item, sem_lini):
        wid = lax.axis_index("s") * NC + lax.axis_index("c")
        base = wid * bpw

        pltpu.sync_copy(item_hbm.at[pl.ds(base, bpw)], item_idx_v)
        item_cp = pltpu.async_copy(table_hbm.at[item_idx_v], item_rows_v,
                                   sem_item)
        lini_cp = pltpu.async_copy(lin_hbm.at[item_idx_v], lin_item_v,
                                   sem_lini)

        def group_body(g, _):
            gflat = (base + g * GB) * LP
            pltpu.sync_copy(ids_hbm.at[pl.ds(gflat, GB * LP)], idx_v)
            pltpu.sync_copy(rat_hbm.at[pl.ds(gflat, GB * LP)], rat_v)

            lin_cps = []
            for s in range(GB * LP // LCHUNK):
                sl = pl.ds(s * LCHUNK, LCHUNK)
                lin_cps.append(
                    pltpu.async_copy(lin_hbm.at[idx_v.at[sl]], linv_v.at[sl],
                                     sem_lin))

            def fire(r):
                return pltpu.async_copy(
                    table_hbm.at[idx_v.at[pl.ds(r * LP, L)]],
                    rows_v.at[r % NBUF], row_sems[r % NBUF])

            cps = {r: fire(r) for r in range(NBUF - 1)}
            for r in range(GB):
                if r + NBUF - 1 < GB:
                    cps[r + NBUF - 1] = fire(r + NBUF - 1)
                cps[r].wait()
                rows = rows_v.at[r % NBUF]
                lbase = r * LP

                def l_body(l, accs, lbase=lbase, rows=rows):
                    rb = plsc.load_gather(
                        rat_v, [jnp.zeros((NLANE,), jnp.int32) + (lbase + l)])
                    return tuple(
                        accs[j] + rb * rows[l, pl.ds(j * NLANE, NLANE)]
                        for j in range(D // NLANE))

                accs = lax.fori_loop(
                    0, L, l_body,
                    tuple(jnp.zeros((NLANE,), jnp.float32)
                          for _ in range(D // NLANE)))
                for j in range(D // NLANE):
                    user_stage[r, pl.ds(j * NLANE, NLANE)] = accs[j]

            for cp in lin_cps:
                cp.wait()

            for c in range(GB // NLANE):
                lanes = (lax.iota(jnp.int32, NLANE) + c * NLANE) * LP

                def lin_body(l, acc, lanes=lanes):
                    lv = plsc.load_gather(linv_v, [lanes + l])
                    rv = plsc.load_gather(rat_v, [lanes + l])
                    return acc + lv * rv

                lin_acc = lax.fori_loop(0, L, lin_body,
                                        jnp.zeros((NLANE,), jnp.float32))
                lin_stage[pl.ds(g * GB + c * NLANE, NLANE)] = lin_acc

            pltpu.sync_copy(user_stage, user_out.at[pl.ds(base + g * GB, GB)])
            return 0

        lax.fori_loop(0, ng, group_body, 0)

        item_cp.wait()
        pltpu.sync_copy(item_rows_v, item_out.at[pl.ds(base, bpw)])
        lini_cp.wait()
        for c in range(bpw // NLANE):
            sl = pl.ds(c * NLANE, NLANE)
            lin_stage[sl] = lin_stage[sl] + lin_item_v[sl]
        pltpu.sync_copy(lin_stage, lin_out.at[pl.ds(base, bpw)])

    return _sc_embed


_sc_embed_half = _make_sc_embed(BH)

BT = 512


def _tc_body(u_ref, i_ref, lin_ref, w0, b0, w1, b1, w2, b2, w3t, bias, o_ref):
    u = u_ref[...]
    it = i_ref[...]
    x = jnp.concatenate([u, it], axis=1)
    h = jnp.maximum(jnp.dot(x, w0[...], preferred_element_type=jnp.float32)
                    + b0[...], 0.0)
    h = jnp.maximum(jnp.dot(h, w1[...], preferred_element_type=jnp.float32)
                    + b1[...], 0.0)
    h = jnp.maximum(jnp.dot(h, w2[...], preferred_element_type=jnp.float32)
                    + b2[...], 0.0)
    mlp = jnp.sum(h * w3t[...], axis=1)
    fm = jnp.sum(u * it, axis=1)
    z = lin_ref[...] + fm + mlp + bias[0, 0]
    o_ref[...] = 1.0 / (1.0 + jnp.exp(-z))


def _tc_mlp(user, item, lin, w0, b0, w1, b1, w2, b2, w3t, bias):
    bh = user.shape[0]
    grid = (bh // BT,)
    full = lambda r, c: pl.BlockSpec((r, c), lambda i: (0, 0))
    return pl.pallas_call(
        _tc_body,
        grid=grid,
        in_specs=[
            pl.BlockSpec((BT, D), lambda i: (i, 0)),
            pl.BlockSpec((BT, D), lambda i: (i, 0)),
            pl.BlockSpec((BT,), lambda i: (i,)),
            full(2 * D, 1024),
            full(1, 1024),
            full(1024, 512),
            full(1, 512),
            full(512, 256),
            full(1, 256),
            full(1, 256),
            pl.BlockSpec(memory_space=pltpu.SMEM),
        ],
        out_specs=pl.BlockSpec((BT,), lambda i: (i,)),
        out_shape=jax.ShapeDtypeStruct((bh,), jnp.float32),
        compiler_params=pltpu.CompilerParams(
            dimension_semantics=("arbitrary",)),
    )(user, item, lin, w0, b0, w1, b1, w2, b2, w3t, bias)


def kernel(feature_ids, feature_ratings, item_ids, emb_table, lin_table,
           lin_bias, W0, b0, W1, b1, W2, b2, W3, b3):
    ids = jnp.pad(feature_ids.astype(jnp.int32),
                  ((0, 0), (0, LP - L))).reshape(-1)
    rat = jnp.pad(feature_ratings, ((0, 0), (0, LP - L))).reshape(-1)
    itm = item_ids.astype(jnp.int32)
    lin_flat = lin_table[:, 0]
    u, i, lin = _sc_embed_half(ids, rat, itm, emb_table, lin_flat)
    bias = (lin_bias + b3).reshape(1, 1)
    return _tc_mlp(u, i, lin, W0, b0.reshape(1, -1), W1, b1.reshape(1, -1),
                   W2, b2.reshape(1, -1), W3.reshape(1, -1), bias)


# speedup vs baseline: 2.1830x; 2.0245x over previous
import functools

import jax
import jax.numpy as jnp
from jax import lax
from jax.experimental import pallas as pl
from jax.experimental.pallas import tpu as pltpu
from jax.experimental.pallas import tpu_sc as plsc

B = 4096
D = 128
L = 50
ST = 56
NC = 2
NS = 16
NW = NC * NS
GB = 32
NBUF = 6
LCHUNK = 80
NLANE = 16
BH = B


def _make_sc_embed(bh):
    bpw = bh // NW
    ng = bpw // GB

    @functools.partial(
        pl.kernel,
        mesh=plsc.VectorSubcoreMesh(core_axis_name="c", subcore_axis_name="s"),
        compiler_params=pltpu.CompilerParams(needs_layout_passes=False),
        out_type=[
            jax.ShapeDtypeStruct((bh, D), jnp.float32),
            jax.ShapeDtypeStruct((bh, D), jnp.float32),
            jax.ShapeDtypeStruct((bh,), jnp.float32),
        ],
        scratch_types=[
            pltpu.VMEM((GB * L + NLANE,), jnp.int32),
            pltpu.VMEM((GB * ST + NLANE,), jnp.int32),
            pltpu.VMEM((GB * L,), jnp.float32),
            pltpu.VMEM((GB * L,), jnp.float32),
            pltpu.VMEM((NBUF, L, D), jnp.float32),
            pltpu.VMEM((bpw,), jnp.int32),
            pltpu.VMEM((bpw, D), jnp.float32),
            pltpu.VMEM((bpw,), jnp.float32),
            pltpu.VMEM((GB, D), jnp.float32),
            pltpu.VMEM((bpw,), jnp.float32),
            [pltpu.SemaphoreType.DMA] * NBUF,
            pltpu.SemaphoreType.DMA,
            pltpu.SemaphoreType.DMA,
            pltpu.SemaphoreType.DMA,
        ],
    )
    def _sc_embed(ids_hbm, rat_hbm, item_hbm, table_hbm, lin_hbm,
                  user_out, item_out, lin_out,
                  idxp_v, idx_v, rat_v, linv_v, rows_v, item_idx_v,
                  item_rows_v, lin_item_v, user_stage, lin_stage,
                  row_sems, sem_lin, sem_item, sem_lini):
        wid = lax.axis_index("s") * NC + lax.axis_index("c")
        base = wid * bpw

        pltpu.sync_copy(item_hbm.at[pl.ds(base, bpw)], item_idx_v)
        item_cp = pltpu.async_copy(table_hbm.at[item_idx_v], item_rows_v,
                                   sem_item)
        lini_cp = pltpu.async_copy(lin_hbm.at[item_idx_v], lin_item_v,
                                   sem_lini)

        def group_body(g, _):
            gflat = (base + g * GB) * L
            pltpu.sync_copy(ids_hbm.at[pl.ds(gflat, GB * L)],
                            idxp_v.at[pl.ds(0, GB * L)])
            pltpu.sync_copy(rat_hbm.at[pl.ds(gflat, GB * L)], rat_v)

            lin_cps = []
            for s in range(GB * L // LCHUNK):
                sl = pl.ds(s * LCHUNK, LCHUNK)
                lin_cps.append(
                    pltpu.async_copy(lin_hbm.at[idxp_v.at[sl]], linv_v.at[sl],
                                     sem_lin))

            for r in range(GB):
                for k in range(0, L, NLANE):
                    idx_v[pl.ds(r * ST + k, NLANE)] = (
                        idxp_v[pl.ds(r * L + k, NLANE)])

            def fire(r):
                return pltpu.async_copy(
                    table_hbm.at[idx_v.at[pl.ds(r * ST, L)]],
                    rows_v.at[r % NBUF], row_sems[r % NBUF])

            cps = {r: fire(r) for r in range(NBUF - 1)}
            for r in range(GB):
                if r + NBUF - 1 < GB:
                    cps[r + NBUF - 1] = fire(r + NBUF - 1)
                cps[r].wait()
                rows = rows_v.at[r % NBUF]
                lbase = r * L

                def l_body(l, accs, lbase=lbase, rows=rows):
                    rb = plsc.load_gather(
                        rat_v, [jnp.zeros((NLANE,), jnp.int32) + (lbase + l)])
                    return tuple(
                        accs[j] + rb * rows[l, pl.ds(j * NLANE, NLANE)]
                        for j in range(D // NLANE))

                accs = lax.fori_loop(
                    0, L, l_body,
                    tuple(jnp.zeros((NLANE,), jnp.float32)
                          for _ in range(D // NLANE)))
                for j in range(D // NLANE):
                    user_stage[r, pl.ds(j * NLANE, NLANE)] = accs[j]

            for cp in lin_cps:
                cp.wait()

            for c in range(GB // NLANE):
                lanes = (lax.iota(jnp.int32, NLANE) + c * NLANE) * L

                def lin_body(l, acc, lanes=lanes):
                    lv = plsc.load_gather(linv_v, [lanes + l])
                    rv = plsc.load_gather(rat_v, [lanes + l])
                    return acc + lv * rv

                lin_acc = lax.fori_loop(0, L, lin_body,
                                        jnp.zeros((NLANE,), jnp.float32))
                lin_stage[pl.ds(g * GB + c * NLANE, NLANE)] = lin_acc

            pltpu.sync_copy(user_stage, user_out.at[pl.ds(base + g * GB, GB)])
            return 0

        lax.fori_loop(0, ng, group_body, 0)

        item_cp.wait()
        pltpu.sync_copy(item_rows_v, item_out.at[pl.ds(base, bpw)])
        lini_cp.wait()
        for c in range(bpw // NLANE):
            sl = pl.ds(c * NLANE, NLANE)
            lin_stage[sl] = lin_stage[sl] + lin_item_v[sl]
        pltpu.sync_copy(lin_stage, lin_out.at[pl.ds(base, bpw)])

    return _sc_embed


_sc_embed_half = _make_sc_embed(BH)

BT = 512


def _tc_body(u_ref, i_ref, lin_ref, w0, b0, w1, b1, w2, b2, w3t, bias, o_ref):
    u = u_ref[...]
    it = i_ref[...]
    x = jnp.concatenate([u, it], axis=1)
    h = jnp.maximum(jnp.dot(x, w0[...], preferred_element_type=jnp.float32)
                    + b0[...], 0.0)
    h = jnp.maximum(jnp.dot(h, w1[...], preferred_element_type=jnp.float32)
                    + b1[...], 0.0)
    h = jnp.maximum(jnp.dot(h, w2[...], preferred_element_type=jnp.float32)
                    + b2[...], 0.0)
    mlp = jnp.sum(h * w3t[...], axis=1)
    fm = jnp.sum(u * it, axis=1)
    z = lin_ref[...] + fm + mlp + bias[0, 0]
    o_ref[...] = 1.0 / (1.0 + jnp.exp(-z))


def _tc_mlp(user, item, lin, w0, b0, w1, b1, w2, b2, w3t, bias):
    bh = user.shape[0]
    grid = (bh // BT,)
    full = lambda r, c: pl.BlockSpec((r, c), lambda i: (0, 0))
    return pl.pallas_call(
        _tc_body,
        grid=grid,
        in_specs=[
            pl.BlockSpec((BT, D), lambda i: (i, 0)),
            pl.BlockSpec((BT, D), lambda i: (i, 0)),
            pl.BlockSpec((BT,), lambda i: (i,)),
            full(2 * D, 1024),
            full(1, 1024),
            full(1024, 512),
            full(1, 512),
            full(512, 256),
            full(1, 256),
            full(1, 256),
            pl.BlockSpec(memory_space=pltpu.SMEM),
        ],
        out_specs=pl.BlockSpec((BT,), lambda i: (i,)),
        out_shape=jax.ShapeDtypeStruct((bh,), jnp.float32),
        compiler_params=pltpu.CompilerParams(
            dimension_semantics=("arbitrary",)),
    )(user, item, lin, w0, b0, w1, b1, w2, b2, w3t, bias)


def kernel(feature_ids, feature_ratings, item_ids, emb_table, lin_table,
           lin_bias, W0, b0, W1, b1, W2, b2, W3, b3):
    ids = feature_ids.astype(jnp.int32).reshape(-1)
    rat = feature_ratings.reshape(-1)
    itm = item_ids.astype(jnp.int32)
    lin_flat = lin_table[:, 0]
    u, i, lin = _sc_embed_half(ids, rat, itm, emb_table, lin_flat)
    bias = (lin_bias + b3).reshape(1, 1)
    return _tc_mlp(u, i, lin, W0, b0.reshape(1, -1), W1, b1.reshape(1, -1),
                   W2, b2.reshape(1, -1), W3.reshape(1, -1), bias)
